# baseline (speedup 1.0000x reference)
import functools

import jax
import jax.numpy as jnp
from jax import lax
from jax.experimental import pallas as pl
from jax.experimental.pallas import tpu as pltpu
from jax.experimental.pallas import tpu_sc as plsc

_LAMBDA = 0.001
_BATCH = 16384
_FACTOR = 16

_info = plsc.get_sparse_core_info()
_NC, _NS, _L = _info.num_cores, _info.num_subcores, _info.num_lanes
_NW = _NC * _NS
_BPW = _BATCH // _NW
_CHUNK = 128
_NCHUNK = _BPW // _CHUNK
_NBLK = _BPW // _L


@functools.partial(
    pl.kernel,
    out_type=jax.ShapeDtypeStruct((_NW, 3 * _L), jnp.float32),
    mesh=plsc.VectorSubcoreMesh(core_axis_name="c", subcore_axis_name="s"),
    compiler_params=pltpu.CompilerParams(
        needs_layout_passes=False, use_tc_tiling_on_sc=False),
    scratch_types=[
        pltpu.VMEM((_NCHUNK, _CHUNK), jnp.int32),
        pltpu.VMEM((_NCHUNK, _CHUNK), jnp.int32),
        pltpu.VMEM((_BPW,), jnp.float32),
        pltpu.VMEM((_BPW, _FACTOR), jnp.float32),
        pltpu.VMEM((_BPW, _FACTOR), jnp.float32),
        pltpu.VMEM((3 * _L,), jnp.float32),
        pltpu.SemaphoreType.DMA,
    ],
)
def _bpr_partials(uidx_hbm, iidx_hbm, rat_hbm, wu_hbm, wi_hbm, out_hbm,
                  uidx_v, iidx_v, rat_v, urows_v, irows_v, out_v, sem):
    wid = lax.axis_index("s") * _NC + lax.axis_index("c")
    pltpu.sync_copy(uidx_hbm.at[wid], uidx_v)
    pltpu.sync_copy(iidx_hbm.at[wid], iidx_v)
    pltpu.sync_copy(rat_hbm.at[wid], rat_v)

    copies = []
    for j in range(_NCHUNK):
        copies.append(pltpu.async_copy(
            wu_hbm.at[uidx_v.at[j]], urows_v.at[pl.ds(j * _CHUNK, _CHUNK)], sem))
        copies.append(pltpu.async_copy(
            wi_hbm.at[iidx_v.at[j]], irows_v.at[pl.ds(j * _CHUNK, _CHUNK)], sem))
    for c in copies:
        c.wait()

    lanes = lax.iota(jnp.int32, _L)

    def blk_body(b, carry):
        task_acc, usq_acc, isq_acc = carry
        rows = b * _L + lanes
        pred = jnp.zeros((_L,), jnp.float32)
        for d in range(_FACTOR):
            cols = jnp.full((_L,), d, jnp.int32)
            u = plsc.load_gather(urows_v, [rows, cols])
            it = plsc.load_gather(irows_v, [rows, cols])
            pred = pred + u * it
            usq_acc = usq_acc + u * u
            isq_acc = isq_acc + it * it
        diff = pred - rat_v[pl.ds(b * _L, _L)]
        task_acc = task_acc + diff * diff
        return task_acc, usq_acc, isq_acc

    z = jnp.zeros((_L,), jnp.float32)
    task_acc, usq_acc, isq_acc = lax.fori_loop(0, _NBLK, blk_body, (z, z, z))
    out_v[pl.ds(0, _L)] = task_acc
    out_v[pl.ds(_L, _L)] = usq_acc
    out_v[pl.ds(2 * _L, _L)] = isq_acc
    pltpu.sync_copy(out_v, out_hbm.at[wid])


def kernel(user0, item_i0, ratings, W_user, W_item):
    uidx = user0.reshape(_NW, _NCHUNK, _CHUNK)
    iidx = item_i0.reshape(_NW, _NCHUNK, _CHUNK)
    rat = ratings.astype(jnp.float32).reshape(_NW, _BPW)
    partials = _bpr_partials(uidx, iidx, rat, W_user, W_item)
    p = partials.reshape(_NW, 3, _L)
    task_loss = p[:, 0, :].sum() / _BATCH
    l2 = _LAMBDA * (p[:, 1, :].sum() + p[:, 2, :].sum()) / (_BATCH * _FACTOR)
    loss = task_loss + l2
    return (loss, task_loss, l2)


# speedup vs baseline: 5.4974x; 5.4974x over previous
import functools

import jax
import jax.numpy as jnp
from jax import lax
from jax.experimental import pallas as pl
from jax.experimental.pallas import tpu as pltpu
from jax.experimental.pallas import tpu_sc as plsc

_LAMBDA = 0.001
_SIZE = 1000000
_BATCH = 16384
_FACTOR = 16

_info = plsc.get_sparse_core_info()
_NC, _NS, _L = _info.num_cores, _info.num_subcores, _info.num_lanes
_NW = _NC * _NS
_BPW = _BATCH // _NW
_NBLK = _BPW // _L
_TAIL = (_SIZE // 128) * 128
_LASTB = _TAIL // 128 - 1


@functools.partial(
    pl.kernel,
    out_type=jax.ShapeDtypeStruct((_NW, 3 * _L), jnp.float32),
    mesh=plsc.VectorSubcoreMesh(core_axis_name="c", subcore_axis_name="s"),
    compiler_params=pltpu.CompilerParams(needs_layout_passes=False),
    scratch_types=[
        pltpu.VMEM((_BPW,), jnp.int32),
        pltpu.VMEM((_BPW,), jnp.int32),
        pltpu.VMEM((_BPW,), jnp.float32),
        pltpu.VMEM((_L, _FACTOR, 128), jnp.float32),
        pltpu.VMEM((_L, _FACTOR, 128), jnp.float32),
        pltpu.VMEM((_FACTOR, 128), jnp.float32),
        pltpu.VMEM((_FACTOR, 128), jnp.float32),
        pltpu.VMEM((3 * _L,), jnp.float32),
        pltpu.SemaphoreType.DMA,
        pltpu.SemaphoreType.DMA,
    ],
)
def _bpr_partials(uidx_hbm, iidx_hbm, rat_hbm, wu_hbm, wi_hbm,
                  wu_tail_hbm, wi_tail_hbm, out_hbm,
                  uidx_v, iidx_v, rat_v, ublk_v, iblk_v,
                  utail_v, itail_v, out_v, semu, semi):
    wid = lax.axis_index("s") * _NC + lax.axis_index("c")
    base = wid * _BPW
    pltpu.sync_copy(uidx_hbm.at[pl.ds(base, _BPW)], uidx_v)
    pltpu.sync_copy(iidx_hbm.at[pl.ds(base, _BPW)], iidx_v)
    pltpu.sync_copy(rat_hbm.at[pl.ds(base, _BPW)], rat_v)
    pltpu.sync_copy(wu_tail_hbm, utail_v)
    pltpu.sync_copy(wi_tail_hbm, itail_v)

    lanes = lax.iota(jnp.int32, _L)

    def fetch(g):
        uvec = uidx_v[pl.ds(g * _L, _L)]
        ivec = iidx_v[pl.ds(g * _L, _L)]
        ub = jnp.minimum(uvec >> 7, jnp.full((_L,), _LASTB, jnp.int32))
        ib = jnp.minimum(ivec >> 7, jnp.full((_L,), _LASTB, jnp.int32))
        copies = []
        for k in range(_L):
            us = pl.multiple_of(ub[k] * 128, 128)
            its = pl.multiple_of(ib[k] * 128, 128)
            copies.append(pltpu.async_copy(
                wu_hbm.at[:, pl.ds(us, 128)], ublk_v.at[k], semu))
            copies.append(pltpu.async_copy(
                wi_hbm.at[:, pl.ds(its, 128)], iblk_v.at[k], semi))
        return uvec, ivec, ub, ib, copies

    def compute(g, uvec, ivec, ub, ib, carry):
        task_acc, usq_acc, isq_acc = carry
        ulane = uvec - ub * 128
        ilane = ivec - ib * 128
        utail_m = uvec >= _TAIL
        itail_m = ivec >= _TAIL
        ulane_t = uvec - _TAIL
        ilane_t = ivec - _TAIL
        pred = jnp.zeros((_L,), jnp.float32)
        for d in range(_FACTOR):
            dsplat = jnp.full((_L,), d, jnp.int32)
            u_m = plsc.load_gather(ublk_v, [lanes, dsplat, ulane])
            i_m = plsc.load_gather(iblk_v, [lanes, dsplat, ilane])
            u_t = plsc.load_gather(utail_v, [dsplat, ulane_t], mask=utail_m)
            i_t = plsc.load_gather(itail_v, [dsplat, ilane_t], mask=itail_m)
            u = jnp.where(utail_m, u_t, u_m)
            it = jnp.where(itail_m, i_t, i_m)
            pred = pred + u * it
            usq_acc = usq_acc + u * u
            isq_acc = isq_acc + it * it
        diff = pred - rat_v[pl.ds(g * _L, _L)]
        task_acc = task_acc + diff * diff
        return task_acc, usq_acc, isq_acc

    def group_body(g, carry):
        uvec, ivec, ub, ib, copies = fetch(g)
        for c in copies:
            c.wait()
        return compute(g, uvec, ivec, ub, ib, carry)

    z = jnp.zeros((_L,), jnp.float32)
    task_acc, usq_acc, isq_acc = lax.fori_loop(0, _NBLK, group_body, (z, z, z))
    out_v[pl.ds(0, _L)] = task_acc
    out_v[pl.ds(_L, _L)] = usq_acc
    out_v[pl.ds(2 * _L, _L)] = isq_acc
    pltpu.sync_copy(out_v, out_hbm.at[wid])


def kernel(user0, item_i0, ratings, W_user, W_item):
    rat = ratings.astype(jnp.float32)
    wu_tail = jnp.zeros((_FACTOR, 128), jnp.float32).at[:, :_SIZE - _TAIL].set(
        W_user[_TAIL:].T)
    wi_tail = jnp.zeros((_FACTOR, 128), jnp.float32).at[:, :_SIZE - _TAIL].set(
        W_item[_TAIL:].T)
    partials = _bpr_partials(user0, item_i0, rat, W_user.T, W_item.T,
                             wu_tail, wi_tail)
    p = partials.reshape(_NW, 3, _L)
    task_loss = p[:, 0, :].sum() / _BATCH
    l2 = _LAMBDA * (p[:, 1, :].sum() + p[:, 2, :].sum()) / (_BATCH * _FACTOR)
    loss = task_loss + l2
    return (loss, task_loss, l2)
